# baseline (speedup 1.0000x reference)
import functools

import jax
import jax.numpy as jnp
from jax import lax
from jax.experimental import pallas as pl
from jax.experimental.pallas import tpu as pltpu
from jax.experimental.pallas import tpu_sc as plsc

NC, NS = 2, 16
NW = NC * NS
L = 16
CB = 2
CW = CB * 128
PER_W = 122
SCALE = 8.0
S = 4096
T = 200
SBLK = S // NW
D = 64


def _t_body(tabT_hbm, tailT_hbm, out_hbm, in_v0, in_v1, ot_v0, ot_v1,
            isem0, isem1, osem0, osem1):
    wid = lax.axis_index("s") * NC + lax.axis_index("c")

    in_vs = (in_v0, in_v1)
    ot_vs = (ot_v0, ot_v1)
    isems = (isem0, isem1)
    osems = (osem0, osem1)

    def fire_read(j0, b):
        for k in range(8):
            pltpu.async_copy(
                tabT_hbm.at[pl.ds(k * 8, 8), pl.ds(j0 * 128, CW)],
                in_vs[b].at[k], isems[b])

    def drain_read(b):
        for k in range(8):
            pltpu.make_async_copy(
                tabT_hbm.at[pl.ds(k * 8, 8), pl.ds(0, CW)],
                in_vs[b].at[k], isems[b]).wait()

    def transpose(b):
        iota = lax.iota(jnp.int32, L)

        @pl.loop(0, CW // L)
        def _grp(g):
            lanes = iota + g * L
            base = lanes * 64
            for k in range(8):
                for r in range(8):
                    d = 8 * k + r
                    val = in_vs[b][k, r, pl.ds(g * L, L)] * SCALE
                    plsc.store_scatter(ot_vs[b], [base + d], val)

    def fire_write(j0, b):
        pltpu.async_copy(
            ot_vs[b], out_hbm.at[pl.ds(j0 * 128 * 64, CW * 64)], osems[b])

    def drain_write(b):
        pltpu.make_async_copy(
            out_hbm.at[pl.ds(0, CW * 64)], ot_vs[b], osems[b]).wait()

    def j0_of(c):
        return (wid * PER_W + c) * CB

    fire_read(j0_of(0), 0)

    @pl.loop(0, PER_W // 2)
    def _pair(p):
        c0 = 2 * p

        @pl.when(p > 0)
        def _():
            drain_write(1)
        fire_read(j0_of(c0 + 1), 1)
        drain_read(0)
        transpose(0)
        fire_write(j0_of(c0), 0)

        @pl.when(p + 1 < PER_W // 2)
        def _():
            drain_write(0)
            fire_read(j0_of(c0 + 2), 0)
        drain_read(1)
        transpose(1)
        fire_write(j0_of(c0 + 1), 1)

    drain_write(0)
    drain_write(1)

    @pl.when(wid == 0)
    def _tail():
        @pl.loop(0, 2)
        def _pairblk(i):
            j = 7808 + i * 2
            fire_read(j, 0)
            drain_read(0)
            transpose(0)
            fire_write(j, 0)
            drain_write(0)
        for k in range(8):
            pltpu.async_copy(
                tailT_hbm.at[pl.ds(k * 8, 8), :],
                in_vs[0].at[k, :, pl.ds(0, 128)], isems[0])
        for k in range(8):
            pltpu.make_async_copy(
                tailT_hbm.at[pl.ds(k * 8, 8), :],
                in_vs[0].at[k, :, pl.ds(0, 128)], isems[0]).wait()
        iota = lax.iota(jnp.int32, L)

        @pl.loop(0, 4)
        def _tgrp(g):
            lanes = iota + g * L
            base = lanes * 64
            for k in range(8):
                for r in range(8):
                    d = 8 * k + r
                    val = in_vs[0][k, r, pl.ds(g * L, L)] * SCALE
                    plsc.store_scatter(ot_vs[0], [base + d], val)
        pltpu.async_copy(
            ot_vs[0].at[pl.ds(0, 64 * 64)],
            out_hbm.at[pl.ds(7812 * 128 * 64, 64 * 64)], osems[0])
        pltpu.make_async_copy(
            out_hbm.at[pl.ds(0, 64 * 64)], ot_vs[0].at[pl.ds(0, 64 * 64)],
            osems[0]).wait()


_transpose = functools.partial(
    pl.kernel,
    out_type=jax.ShapeDtypeStruct((64000000,), jnp.float32),
    mesh=plsc.VectorSubcoreMesh(core_axis_name="c", subcore_axis_name="s"),
    scratch_types=[
        pltpu.VMEM((8, 8, CW), jnp.float32),
        pltpu.VMEM((8, 8, CW), jnp.float32),
        pltpu.VMEM((CW * 64,), jnp.float32),
        pltpu.VMEM((CW * 64,), jnp.float32),
        pltpu.SemaphoreType.DMA,
        pltpu.SemaphoreType.DMA,
        pltpu.SemaphoreType.DMA,
        pltpu.SemaphoreType.DMA,
    ],
    compiler_params=pltpu.CompilerParams(needs_layout_passes=False),
)(_t_body)


def _g_body(xt_hbm, tab_hbm, out_hbm, idxs_v, rows_v0, rows_v1,
            gsem0, gsem1, osem0, osem1):
    wid = lax.axis_index("s") * NC + lax.axis_index("c")
    pltpu.sync_copy(xt_hbm.at[:, pl.ds(wid * SBLK, SBLK)], idxs_v)

    rows = (rows_v0, rows_v1)
    gsems = (gsem0, gsem1)
    osems = (osem0, osem1)

    def fire_gather(t, b):
        pltpu.async_copy(tab_hbm.at[idxs_v.at[t]], rows[b], gsems[b])

    def drain_gather(b):
        pltpu.make_async_copy(
            tab_hbm.at[pl.ds(0, SBLK)], rows[b], gsems[b]).wait()

    def fire_out(t, b):
        pltpu.async_copy(
            rows[b],
            out_hbm.at[pl.ds(wid * SBLK, SBLK), t, pl.ds(0, D)],
            osems[b])

    def drain_out(b):
        pltpu.make_async_copy(
            out_hbm.at[pl.ds(0, SBLK), 0, pl.ds(0, D)], rows[b],
            osems[b]).wait()

    fire_gather(0, 0)

    @pl.loop(0, T // 2)
    def _pair(p):
        t0 = 2 * p

        @pl.when(p > 0)
        def _():
            drain_out(1)
        fire_gather(t0 + 1, 1)
        drain_gather(0)
        fire_out(t0, 0)

        @pl.when(p + 1 < T // 2)
        def _():
            drain_out(0)
            fire_gather(t0 + 2, 0)
        drain_gather(1)
        fire_out(t0 + 1, 1)

    drain_out(0)
    drain_out(1)


_gather = functools.partial(
    pl.kernel,
    out_type=jax.ShapeDtypeStruct((S, T, 128), jnp.float32),
    mesh=plsc.VectorSubcoreMesh(core_axis_name="c", subcore_axis_name="s"),
    scratch_types=[
        pltpu.VMEM((T, SBLK), jnp.int32),
        pltpu.VMEM((SBLK, D), jnp.float32),
        pltpu.VMEM((SBLK, D), jnp.float32),
        pltpu.SemaphoreType.DMA,
        pltpu.SemaphoreType.DMA,
        pltpu.SemaphoreType.DMA,
        pltpu.SemaphoreType.DMA,
    ],
    compiler_params=pltpu.CompilerParams(use_tc_tiling_on_sc=False),
)(_g_body)


def kernel(x, table):
    tail = jnp.pad(table[999936:], ((0, 64), (0, 0))).T
    tab_lin = _transpose(table.T, tail).reshape(1000000, 64)
    o3 = _gather(x.T, tab_lin)
    return o3[:, :, :D]


# speedup vs baseline: 1.3135x; 1.3135x over previous
import functools

import jax
import jax.numpy as jnp
from jax import lax
from jax.experimental import pallas as pl
from jax.experimental.pallas import tpu as pltpu
from jax.experimental.pallas import tpu_sc as plsc

NC, NS = 2, 16
NW = NC * NS
L = 16
CB = 2
CW = CB * 128
PER_W = 122
SCALE = 8.0
S = 4096
T = 200
SBLK = S // NW
D = 64


def _t_body(tabT_hbm, tailT_hbm, out_hbm, in_v0, in_v1, ot_v0, ot_v1,
            isem0, isem1, osem0, osem1):
    wid = lax.axis_index("s") * NC + lax.axis_index("c")

    in_vs = (in_v0, in_v1)
    ot_vs = (ot_v0, ot_v1)
    isems = (isem0, isem1)
    osems = (osem0, osem1)

    def fire_read(j0, b):
        for k in range(8):
            pltpu.async_copy(
                tabT_hbm.at[pl.ds(k * 8, 8), pl.ds(j0 * 128, CW)],
                in_vs[b].at[k], isems[b])

    def drain_read(b):
        for k in range(8):
            pltpu.make_async_copy(
                tabT_hbm.at[pl.ds(k * 8, 8), pl.ds(0, CW)],
                in_vs[b].at[k], isems[b]).wait()

    def transpose(b):
        iota = lax.iota(jnp.int32, L)

        @pl.loop(0, CW // L)
        def _grp(g):
            lanes = iota + g * L
            base = lanes * 64
            for k in range(8):
                vals = [in_vs[b][k, r, pl.ds(g * L, L)] * SCALE
                        for r in range(8)]
                for r in range(8):
                    plsc.store_scatter(ot_vs[b], [base + 8 * k + r], vals[r])

    def fire_write(j0, b):
        pltpu.async_copy(
            ot_vs[b], out_hbm.at[pl.ds(j0 * 128 * 64, CW * 64)], osems[b])

    def drain_write(b):
        pltpu.make_async_copy(
            out_hbm.at[pl.ds(0, CW * 64)], ot_vs[b], osems[b]).wait()

    def j0_of(c):
        return (wid * PER_W + c) * CB

    fire_read(j0_of(0), 0)

    @pl.loop(0, PER_W // 2)
    def _pair(p):
        c0 = 2 * p

        @pl.when(p > 0)
        def _():
            drain_write(1)
        fire_read(j0_of(c0 + 1), 1)
        drain_read(0)
        transpose(0)
        fire_write(j0_of(c0), 0)

        @pl.when(p + 1 < PER_W // 2)
        def _():
            drain_write(0)
            fire_read(j0_of(c0 + 2), 0)
        drain_read(1)
        transpose(1)
        fire_write(j0_of(c0 + 1), 1)

    drain_write(0)
    drain_write(1)

    @pl.when(wid == 0)
    def _tail():
        @pl.loop(0, 2)
        def _pairblk(i):
            j = 7808 + i * 2
            fire_read(j, 0)
            drain_read(0)
            transpose(0)
            fire_write(j, 0)
            drain_write(0)
        for k in range(8):
            pltpu.async_copy(
                tailT_hbm.at[pl.ds(k * 8, 8), :],
                in_vs[0].at[k, :, pl.ds(0, 128)], isems[0])
        for k in range(8):
            pltpu.make_async_copy(
                tailT_hbm.at[pl.ds(k * 8, 8), :],
                in_vs[0].at[k, :, pl.ds(0, 128)], isems[0]).wait()
        iota = lax.iota(jnp.int32, L)

        @pl.loop(0, 4)
        def _tgrp(g):
            lanes = iota + g * L
            base = lanes * 64
            for k in range(8):
                vals = [in_vs[0][k, r, pl.ds(g * L, L)] * SCALE
                        for r in range(8)]
                for r in range(8):
                    plsc.store_scatter(ot_vs[0], [base + 8 * k + r], vals[r])
        pltpu.async_copy(
            ot_vs[0].at[pl.ds(0, 64 * 64)],
            out_hbm.at[pl.ds(7812 * 128 * 64, 64 * 64)], osems[0])
        pltpu.make_async_copy(
            out_hbm.at[pl.ds(0, 64 * 64)], ot_vs[0].at[pl.ds(0, 64 * 64)],
            osems[0]).wait()


_transpose = functools.partial(
    pl.kernel,
    out_type=jax.ShapeDtypeStruct((64000000,), jnp.float32),
    mesh=plsc.VectorSubcoreMesh(core_axis_name="c", subcore_axis_name="s"),
    scratch_types=[
        pltpu.VMEM((8, 8, CW), jnp.float32),
        pltpu.VMEM((8, 8, CW), jnp.float32),
        pltpu.VMEM((CW * 64,), jnp.float32),
        pltpu.VMEM((CW * 64,), jnp.float32),
        pltpu.SemaphoreType.DMA,
        pltpu.SemaphoreType.DMA,
        pltpu.SemaphoreType.DMA,
        pltpu.SemaphoreType.DMA,
    ],
    compiler_params=pltpu.CompilerParams(needs_layout_passes=False, disable_bounds_checks=True),
)(_t_body)


def _g_body(xt_hbm, tab_hbm, out_hbm, idxs_v, rows_v0, rows_v1,
            gsem0, gsem1, osem0, osem1):
    wid = lax.axis_index("s") * NC + lax.axis_index("c")
    pltpu.sync_copy(xt_hbm.at[:, pl.ds(wid * SBLK, SBLK)], idxs_v)

    rows = (rows_v0, rows_v1)
    gsems = (gsem0, gsem1)
    osems = (osem0, osem1)

    def fire_gather(t, b):
        pltpu.async_copy(tab_hbm.at[idxs_v.at[t]], rows[b], gsems[b])

    def drain_gather(b):
        pltpu.make_async_copy(
            tab_hbm.at[pl.ds(0, SBLK)], rows[b], gsems[b]).wait()

    def fire_out(t, b):
        pltpu.async_copy(
            rows[b],
            out_hbm.at[pl.ds(wid * SBLK, SBLK), t, pl.ds(0, D)],
            osems[b])

    def drain_out(b):
        pltpu.make_async_copy(
            out_hbm.at[pl.ds(0, SBLK), 0, pl.ds(0, D)], rows[b],
            osems[b]).wait()

    fire_gather(0, 0)

    @pl.loop(0, T // 2)
    def _pair(p):
        t0 = 2 * p

        @pl.when(p > 0)
        def _():
            drain_out(1)
        fire_gather(t0 + 1, 1)
        drain_gather(0)
        fire_out(t0, 0)

        @pl.when(p + 1 < T // 2)
        def _():
            drain_out(0)
            fire_gather(t0 + 2, 0)
        drain_gather(1)
        fire_out(t0 + 1, 1)

    drain_out(0)
    drain_out(1)


_gather = functools.partial(
    pl.kernel,
    out_type=jax.ShapeDtypeStruct((S, T, 128), jnp.float32),
    mesh=plsc.VectorSubcoreMesh(core_axis_name="c", subcore_axis_name="s"),
    scratch_types=[
        pltpu.VMEM((T, SBLK), jnp.int32),
        pltpu.VMEM((SBLK, D), jnp.float32),
        pltpu.VMEM((SBLK, D), jnp.float32),
        pltpu.SemaphoreType.DMA,
        pltpu.SemaphoreType.DMA,
        pltpu.SemaphoreType.DMA,
        pltpu.SemaphoreType.DMA,
    ],
    compiler_params=pltpu.CompilerParams(use_tc_tiling_on_sc=False, disable_bounds_checks=True),
)(_g_body)


def kernel(x, table):
    tail = jnp.pad(table[999936:], ((0, 64), (0, 0))).T
    tab_lin = _transpose(table.T, tail).reshape(1000000, 64)
    o3 = _gather(x.T, tab_lin)
    return o3[:, :, :D]


# speedup vs baseline: 1.3218x; 1.0063x over previous
import functools

import jax
import jax.numpy as jnp
from jax import lax
from jax.experimental import pallas as pl
from jax.experimental.pallas import tpu as pltpu
from jax.experimental.pallas import tpu_sc as plsc

NC, NS = 2, 16
NW = NC * NS
L = 16
CB = 2
CW = CB * 128
PER_W = 122
SCALE = 8.0
S = 4096
T = 200
SBLK = S // NW
D = 64


def _t_body(tabT_hbm, tailT_hbm, out_hbm, in_v0, in_v1, ot_v0, ot_v1,
            isem0, isem1, osem0, osem1):
    wid = lax.axis_index("s") * NC + lax.axis_index("c")

    in_vs = (in_v0, in_v1)
    ot_vs = (ot_v0, ot_v1)
    isems = (isem0, isem1)
    osems = (osem0, osem1)

    def fire_read(j0, b):
        for k in range(8):
            pltpu.async_copy(
                tabT_hbm.at[pl.ds(k * 8, 8), pl.ds(j0 * 128, CW)],
                in_vs[b].at[k], isems[b])

    def drain_read(b):
        pltpu.make_async_copy(
            tabT_hbm.at[pl.ds(0, 8), pl.ds(0, CW * 8)],
            in_vs[b], isems[b]).wait()

    def transpose(b):
        iota = lax.iota(jnp.int32, L)

        @pl.loop(0, CW // L)
        def _grp(g):
            lanes = iota + g * L
            base = lanes * 64
            for k in range(8):
                vals = [in_vs[b][k, r, pl.ds(g * L, L)] * SCALE
                        for r in range(8)]
                for r in range(8):
                    plsc.store_scatter(ot_vs[b], [base + 8 * k + r], vals[r])

    def fire_write(j0, b):
        pltpu.async_copy(
            ot_vs[b], out_hbm.at[pl.ds(j0 * 128 * 64, CW * 64)], osems[b])

    def drain_write(b):
        pltpu.make_async_copy(
            out_hbm.at[pl.ds(0, CW * 64)], ot_vs[b], osems[b]).wait()

    def j0_of(c):
        return (wid * PER_W + c) * CB

    fire_read(j0_of(0), 0)

    @pl.loop(0, PER_W // 2)
    def _pair(p):
        c0 = 2 * p

        @pl.when(p > 0)
        def _():
            drain_write(1)
        fire_read(j0_of(c0 + 1), 1)
        drain_read(0)
        transpose(0)
        fire_write(j0_of(c0), 0)

        @pl.when(p + 1 < PER_W // 2)
        def _():
            drain_write(0)
            fire_read(j0_of(c0 + 2), 0)
        drain_read(1)
        transpose(1)
        fire_write(j0_of(c0 + 1), 1)

    drain_write(0)
    drain_write(1)

    @pl.when(wid == 0)
    def _tail():
        @pl.loop(0, 2)
        def _pairblk(i):
            j = 7808 + i * 2
            fire_read(j, 0)
            drain_read(0)
            transpose(0)
            fire_write(j, 0)
            drain_write(0)
        for k in range(8):
            pltpu.async_copy(
                tailT_hbm.at[pl.ds(k * 8, 8), :],
                in_vs[0].at[k, :, pl.ds(0, 128)], isems[0])
        for k in range(8):
            pltpu.make_async_copy(
                tailT_hbm.at[pl.ds(k * 8, 8), :],
                in_vs[0].at[k, :, pl.ds(0, 128)], isems[0]).wait()
        iota = lax.iota(jnp.int32, L)

        @pl.loop(0, 4)
        def _tgrp(g):
            lanes = iota + g * L
            base = lanes * 64
            for k in range(8):
                vals = [in_vs[0][k, r, pl.ds(g * L, L)] * SCALE
                        for r in range(8)]
                for r in range(8):
                    plsc.store_scatter(ot_vs[0], [base + 8 * k + r], vals[r])
        pltpu.async_copy(
            ot_vs[0].at[pl.ds(0, 64 * 64)],
            out_hbm.at[pl.ds(7812 * 128 * 64, 64 * 64)], osems[0])
        pltpu.make_async_copy(
            out_hbm.at[pl.ds(0, 64 * 64)], ot_vs[0].at[pl.ds(0, 64 * 64)],
            osems[0]).wait()


_transpose = functools.partial(
    pl.kernel,
    out_type=jax.ShapeDtypeStruct((64000000,), jnp.float32),
    mesh=plsc.VectorSubcoreMesh(core_axis_name="c", subcore_axis_name="s"),
    scratch_types=[
        pltpu.VMEM((8, 8, CW), jnp.float32),
        pltpu.VMEM((8, 8, CW), jnp.float32),
        pltpu.VMEM((CW * 64,), jnp.float32),
        pltpu.VMEM((CW * 64,), jnp.float32),
        pltpu.SemaphoreType.DMA,
        pltpu.SemaphoreType.DMA,
        pltpu.SemaphoreType.DMA,
        pltpu.SemaphoreType.DMA,
    ],
    compiler_params=pltpu.CompilerParams(needs_layout_passes=False, disable_bounds_checks=True),
)(_t_body)


def _g_body(xt_hbm, tab_hbm, out_hbm, idxs_v, rows_v0, rows_v1,
            gsem0, gsem1, osem0, osem1):
    wid = lax.axis_index("s") * NC + lax.axis_index("c")
    pltpu.sync_copy(xt_hbm.at[:, pl.ds(wid * SBLK, SBLK)], idxs_v)

    rows = (rows_v0, rows_v1)
    gsems = (gsem0, gsem1)
    osems = (osem0, osem1)

    def fire_gather(t, b):
        pltpu.async_copy(tab_hbm.at[idxs_v.at[t]], rows[b], gsems[b])

    def drain_gather(b):
        pltpu.make_async_copy(
            tab_hbm.at[pl.ds(0, SBLK)], rows[b], gsems[b]).wait()

    def fire_out(t, b):
        pltpu.async_copy(
            rows[b],
            out_hbm.at[pl.ds(wid * SBLK, SBLK), t, pl.ds(0, D)],
            osems[b])

    def drain_out(b):
        pltpu.make_async_copy(
            out_hbm.at[pl.ds(0, SBLK), 0, pl.ds(0, D)], rows[b],
            osems[b]).wait()

    fire_gather(0, 0)

    @pl.loop(0, T // 2)
    def _pair(p):
        t0 = 2 * p

        @pl.when(p > 0)
        def _():
            drain_out(1)
        fire_gather(t0 + 1, 1)
        drain_gather(0)
        fire_out(t0, 0)

        @pl.when(p + 1 < T // 2)
        def _():
            drain_out(0)
            fire_gather(t0 + 2, 0)
        drain_gather(1)
        fire_out(t0 + 1, 1)

    drain_out(0)
    drain_out(1)


_gather = functools.partial(
    pl.kernel,
    out_type=jax.ShapeDtypeStruct((S, T, 128), jnp.float32),
    mesh=plsc.VectorSubcoreMesh(core_axis_name="c", subcore_axis_name="s"),
    scratch_types=[
        pltpu.VMEM((T, SBLK), jnp.int32),
        pltpu.VMEM((SBLK, D), jnp.float32),
        pltpu.VMEM((SBLK, D), jnp.float32),
        pltpu.SemaphoreType.DMA,
        pltpu.SemaphoreType.DMA,
        pltpu.SemaphoreType.DMA,
        pltpu.SemaphoreType.DMA,
    ],
    compiler_params=pltpu.CompilerParams(use_tc_tiling_on_sc=False, disable_bounds_checks=True),
)(_g_body)


def kernel(x, table):
    tail = jnp.pad(table[999936:], ((0, 64), (0, 0))).T
    tab_lin = _transpose(table.T, tail).reshape(1000000, 64)
    o3 = _gather(x.T, tab_lin)
    return o3[:, :, :D]


# speedup vs baseline: 1.4281x; 1.0805x over previous
import functools

import jax
import jax.numpy as jnp
from jax import lax
from jax.experimental import pallas as pl
from jax.experimental.pallas import tpu as pltpu
from jax.experimental.pallas import tpu_sc as plsc

NC, NS = 2, 16
NW = NC * NS
L = 16
CB = 2
CW = CB * 128
PER_W = 122
SCALE = 8.0
S = 4096
T = 200
SBLK = S // NW
D = 64


def _t_body(tabT_hbm, tailT_hbm, out_hbm, in_v0, in_v1, ot_v0, ot_v1,
            isem0, isem1, osem0, osem1):
    wid = lax.axis_index("s") * NC + lax.axis_index("c")

    in_vs = (in_v0, in_v1)
    ot_vs = (ot_v0, ot_v1)
    isems = (isem0, isem1)
    osems = (osem0, osem1)

    def fire_read(j0, b):
        for k in range(8):
            pltpu.async_copy(
                tabT_hbm.at[pl.ds(k * 8, 8), pl.ds(j0 * 128, CW)],
                in_vs[b].at[k], isems[b])

    def drain_read(b):
        pltpu.make_async_copy(
            tabT_hbm.at[pl.ds(0, 8), pl.ds(0, CW * 8)],
            in_vs[b], isems[b]).wait()

    def transpose(b):
        iota = lax.iota(jnp.int32, L)

        @pl.loop(0, CW // L)
        def _grp(g):
            lanes = iota + g * L
            base = lanes * 64
            for k in range(8):
                vals = [in_vs[b][k, r, pl.ds(g * L, L)] * SCALE
                        for r in range(8)]
                for r in range(8):
                    plsc.store_scatter(ot_vs[b], [base + 8 * k + r], vals[r])

    def fire_write(j0, b):
        pltpu.async_copy(
            ot_vs[b], out_hbm.at[pl.ds(j0 * 128 * 64, CW * 64)], osems[b])

    def drain_write(b):
        pltpu.make_async_copy(
            out_hbm.at[pl.ds(0, CW * 64)], ot_vs[b], osems[b]).wait()

    def j0_of(c):
        return (wid * PER_W + c) * CB

    fire_read(j0_of(0), 0)

    @pl.loop(0, PER_W // 2)
    def _pair(p):
        c0 = 2 * p

        @pl.when(p > 0)
        def _():
            drain_write(1)
        fire_read(j0_of(c0 + 1), 1)
        drain_read(0)
        transpose(0)
        fire_write(j0_of(c0), 0)

        @pl.when(p + 1 < PER_W // 2)
        def _():
            drain_write(0)
            fire_read(j0_of(c0 + 2), 0)
        drain_read(1)
        transpose(1)
        fire_write(j0_of(c0 + 1), 1)

    drain_write(0)
    drain_write(1)

    @pl.when(wid == 0)
    def _tail():
        @pl.loop(0, 2)
        def _pairblk(i):
            j = 7808 + i * 2
            fire_read(j, 0)
            drain_read(0)
            transpose(0)
            fire_write(j, 0)
            drain_write(0)
        for k in range(8):
            pltpu.async_copy(
                tailT_hbm.at[pl.ds(k * 8, 8), :],
                in_vs[0].at[k, :, pl.ds(0, 128)], isems[0])
        for k in range(8):
            pltpu.make_async_copy(
                tailT_hbm.at[pl.ds(k * 8, 8), :],
                in_vs[0].at[k, :, pl.ds(0, 128)], isems[0]).wait()
        iota = lax.iota(jnp.int32, L)

        @pl.loop(0, 4)
        def _tgrp(g):
            lanes = iota + g * L
            base = lanes * 64
            for k in range(8):
                vals = [in_vs[0][k, r, pl.ds(g * L, L)] * SCALE
                        for r in range(8)]
                for r in range(8):
                    plsc.store_scatter(ot_vs[0], [base + 8 * k + r], vals[r])
        pltpu.async_copy(
            ot_vs[0].at[pl.ds(0, 64 * 64)],
            out_hbm.at[pl.ds(7812 * 128 * 64, 64 * 64)], osems[0])
        pltpu.make_async_copy(
            out_hbm.at[pl.ds(0, 64 * 64)], ot_vs[0].at[pl.ds(0, 64 * 64)],
            osems[0]).wait()


_transpose = functools.partial(
    pl.kernel,
    out_type=jax.ShapeDtypeStruct((64000000,), jnp.float32),
    mesh=plsc.VectorSubcoreMesh(core_axis_name="c", subcore_axis_name="s"),
    scratch_types=[
        pltpu.VMEM((8, 8, CW), jnp.float32),
        pltpu.VMEM((8, 8, CW), jnp.float32),
        pltpu.VMEM((CW * 64,), jnp.float32),
        pltpu.VMEM((CW * 64,), jnp.float32),
        pltpu.SemaphoreType.DMA,
        pltpu.SemaphoreType.DMA,
        pltpu.SemaphoreType.DMA,
        pltpu.SemaphoreType.DMA,
    ],
    compiler_params=pltpu.CompilerParams(needs_layout_passes=False, disable_bounds_checks=True),
)(_t_body)


def _g_body(xt_hbm, tab_hbm, out_hbm, idxs_v, rows_v0, rows_v1,
            gsem0, gsem1, osem0, osem1):
    wid = lax.axis_index("s") * NC + lax.axis_index("c")
    pltpu.sync_copy(xt_hbm.at[:, pl.ds(wid * SBLK, SBLK)], idxs_v)

    rows = (rows_v0, rows_v1)
    gsems = (gsem0, gsem1)
    osems = (osem0, osem1)

    def fire_gather(t, b):
        pltpu.async_copy(tab_hbm.at[idxs_v.at[t]], rows[b], gsems[b])

    def drain_gather(b):
        pltpu.make_async_copy(
            tab_hbm.at[pl.ds(0, SBLK)], rows[b], gsems[b]).wait()

    def fire_out(t, b):
        pltpu.async_copy(
            rows[b],
            out_hbm.at[pl.ds(wid * SBLK, SBLK), t, pl.ds(0, D)],
            osems[b])

    def drain_out(b):
        pltpu.make_async_copy(
            out_hbm.at[pl.ds(0, SBLK), 0, pl.ds(0, D)], rows[b],
            osems[b]).wait()

    fire_gather(0, 0)

    @pl.loop(0, T // 2)
    def _pair(p):
        t0 = 2 * p

        @pl.when(p > 0)
        def _():
            drain_out(1)
        fire_gather(t0 + 1, 1)
        drain_gather(0)
        fire_out(t0, 0)

        @pl.when(p + 1 < T // 2)
        def _():
            drain_out(0)
            fire_gather(t0 + 2, 0)
        drain_gather(1)
        fire_out(t0 + 1, 1)

    drain_out(0)
    drain_out(1)


_gather = functools.partial(
    pl.kernel,
    out_type=jax.ShapeDtypeStruct((S, T, 128), jnp.float32),
    mesh=plsc.VectorSubcoreMesh(core_axis_name="c", subcore_axis_name="s"),
    scratch_types=[
        pltpu.VMEM((T, SBLK), jnp.int32),
        pltpu.VMEM((SBLK, D), jnp.float32),
        pltpu.VMEM((SBLK, D), jnp.float32),
        pltpu.SemaphoreType.DMA,
        pltpu.SemaphoreType.DMA,
        pltpu.SemaphoreType.DMA,
        pltpu.SemaphoreType.DMA,
    ],
    compiler_params=pltpu.CompilerParams(use_tc_tiling_on_sc=False, disable_bounds_checks=True),
)(_g_body)


def kernel(x, table):
    o3 = _gather(x.T, table * SCALE)
    return o3[:, :, :D]


# speedup vs baseline: 1.8066x; 1.2650x over previous
import functools

import jax
import jax.numpy as jnp
from jax import lax
from jax.experimental import pallas as pl
from jax.experimental.pallas import tpu as pltpu
from jax.experimental.pallas import tpu_sc as plsc

NC, NS = 2, 16
NW = NC * NS
L = 16
CB = 2
CW = CB * 128
PER_W = 122
SCALE = 8.0
S = 4096
T = 200
SBLK = S // NW
D = 64


def _t_body(tabT_hbm, tailT_hbm, out_hbm, in_v0, in_v1, ot_v0, ot_v1,
            isem0, isem1, osem0, osem1):
    wid = lax.axis_index("s") * NC + lax.axis_index("c")

    in_vs = (in_v0, in_v1)
    ot_vs = (ot_v0, ot_v1)
    isems = (isem0, isem1)
    osems = (osem0, osem1)

    def fire_read(j0, b):
        for k in range(8):
            pltpu.async_copy(
                tabT_hbm.at[pl.ds(k * 8, 8), pl.ds(j0 * 128, CW)],
                in_vs[b].at[k], isems[b])

    def drain_read(b):
        pltpu.make_async_copy(
            tabT_hbm.at[pl.ds(0, 8), pl.ds(0, CW * 8)],
            in_vs[b], isems[b]).wait()

    def transpose(b):
        iota = lax.iota(jnp.int32, L)

        @pl.loop(0, CW // L)
        def _grp(g):
            lanes = iota + g * L
            base = lanes * 64
            for k in range(8):
                vals = [in_vs[b][k, r, pl.ds(g * L, L)] * SCALE
                        for r in range(8)]
                for r in range(8):
                    plsc.store_scatter(ot_vs[b], [base + 8 * k + r], vals[r])

    def fire_write(j0, b):
        pltpu.async_copy(
            ot_vs[b], out_hbm.at[pl.ds(j0 * 128 * 64, CW * 64)], osems[b])

    def drain_write(b):
        pltpu.make_async_copy(
            out_hbm.at[pl.ds(0, CW * 64)], ot_vs[b], osems[b]).wait()

    def j0_of(c):
        return (wid * PER_W + c) * CB

    fire_read(j0_of(0), 0)

    @pl.loop(0, PER_W // 2)
    def _pair(p):
        c0 = 2 * p

        @pl.when(p > 0)
        def _():
            drain_write(1)
        fire_read(j0_of(c0 + 1), 1)
        drain_read(0)
        transpose(0)
        fire_write(j0_of(c0), 0)

        @pl.when(p + 1 < PER_W // 2)
        def _():
            drain_write(0)
            fire_read(j0_of(c0 + 2), 0)
        drain_read(1)
        transpose(1)
        fire_write(j0_of(c0 + 1), 1)

    drain_write(0)
    drain_write(1)

    @pl.when(wid == 0)
    def _tail():
        @pl.loop(0, 2)
        def _pairblk(i):
            j = 7808 + i * 2
            fire_read(j, 0)
            drain_read(0)
            transpose(0)
            fire_write(j, 0)
            drain_write(0)
        for k in range(8):
            pltpu.async_copy(
                tailT_hbm.at[pl.ds(k * 8, 8), :],
                in_vs[0].at[k, :, pl.ds(0, 128)], isems[0])
        for k in range(8):
            pltpu.make_async_copy(
                tailT_hbm.at[pl.ds(k * 8, 8), :],
                in_vs[0].at[k, :, pl.ds(0, 128)], isems[0]).wait()
        iota = lax.iota(jnp.int32, L)

        @pl.loop(0, 4)
        def _tgrp(g):
            lanes = iota + g * L
            base = lanes * 64
            for k in range(8):
                vals = [in_vs[0][k, r, pl.ds(g * L, L)] * SCALE
                        for r in range(8)]
                for r in range(8):
                    plsc.store_scatter(ot_vs[0], [base + 8 * k + r], vals[r])
        pltpu.async_copy(
            ot_vs[0].at[pl.ds(0, 64 * 64)],
            out_hbm.at[pl.ds(7812 * 128 * 64, 64 * 64)], osems[0])
        pltpu.make_async_copy(
            out_hbm.at[pl.ds(0, 64 * 64)], ot_vs[0].at[pl.ds(0, 64 * 64)],
            osems[0]).wait()


_transpose = functools.partial(
    pl.kernel,
    out_type=jax.ShapeDtypeStruct((64000000,), jnp.float32),
    mesh=plsc.VectorSubcoreMesh(core_axis_name="c", subcore_axis_name="s"),
    scratch_types=[
        pltpu.VMEM((8, 8, CW), jnp.float32),
        pltpu.VMEM((8, 8, CW), jnp.float32),
        pltpu.VMEM((CW * 64,), jnp.float32),
        pltpu.VMEM((CW * 64,), jnp.float32),
        pltpu.SemaphoreType.DMA,
        pltpu.SemaphoreType.DMA,
        pltpu.SemaphoreType.DMA,
        pltpu.SemaphoreType.DMA,
    ],
    compiler_params=pltpu.CompilerParams(needs_layout_passes=False, disable_bounds_checks=True),
)(_t_body)


def _g_body(xt_hbm, tab_hbm, out_hbm, idxs_v, rows_v0, rows_v1,
            gsem0, gsem1, osem0, osem1):
    wid = lax.axis_index("s") * NC + lax.axis_index("c")
    pltpu.sync_copy(xt_hbm.at[:, pl.ds(wid * SBLK, SBLK)], idxs_v)

    rows = (rows_v0, rows_v1)
    gsems = (gsem0, gsem1)
    osems = (osem0, osem1)

    def fire_gather(t, b):
        pltpu.async_copy(tab_hbm.at[idxs_v.at[t]], rows[b], gsems[b])

    def drain_gather(b):
        pltpu.make_async_copy(
            tab_hbm.at[pl.ds(0, SBLK)], rows[b], gsems[b]).wait()

    def fire_out(t, b):
        pltpu.async_copy(
            rows[b],
            out_hbm.at[pl.ds(wid * SBLK, SBLK), t, pl.ds(0, D)],
            osems[b])

    def drain_out(b):
        pltpu.make_async_copy(
            out_hbm.at[pl.ds(0, SBLK), 0, pl.ds(0, D)], rows[b],
            osems[b]).wait()

    def scale(b):
        @pl.loop(0, SBLK)
        def _row(r):
            for j in range(D // L):
                sl = pl.ds(j * L, L)
                rows[b][r, sl] = rows[b][r, sl] * SCALE

    fire_gather(0, 0)

    @pl.loop(0, T // 2)
    def _pair(p):
        t0 = 2 * p

        @pl.when(p > 0)
        def _():
            drain_out(1)
        fire_gather(t0 + 1, 1)
        drain_gather(0)
        scale(0)
        fire_out(t0, 0)

        @pl.when(p + 1 < T // 2)
        def _():
            drain_out(0)
            fire_gather(t0 + 2, 0)
        drain_gather(1)
        scale(1)
        fire_out(t0 + 1, 1)

    drain_out(0)
    drain_out(1)


_gather = functools.partial(
    pl.kernel,
    out_type=jax.ShapeDtypeStruct((S, T, 128), jnp.float32),
    mesh=plsc.VectorSubcoreMesh(core_axis_name="c", subcore_axis_name="s"),
    scratch_types=[
        pltpu.VMEM((T, SBLK), jnp.int32),
        pltpu.VMEM((SBLK, D), jnp.float32),
        pltpu.VMEM((SBLK, D), jnp.float32),
        pltpu.SemaphoreType.DMA,
        pltpu.SemaphoreType.DMA,
        pltpu.SemaphoreType.DMA,
        pltpu.SemaphoreType.DMA,
    ],
    compiler_params=pltpu.CompilerParams(use_tc_tiling_on_sc=False, disable_bounds_checks=True),
)(_g_body)


def kernel(x, table):
    o3 = _gather(x.T, table)
    return o3[:, :, :D]
